# baseline (speedup 1.0000x reference)
import functools

import jax
import jax.numpy as jnp
from jax import lax
from jax.experimental import pallas as pl
from jax.experimental.pallas import tpu as pltpu
from jax.experimental.pallas import tpu_sc as plsc

NC = 2
NS = 16
NW = NC * NS
CHS = 8
NLANES = 16
TC_BS = 256


def _sc_body(B, S, D, s_lo, s_hi, x_hbm, pos_hbm, o_hbm,
             bx0, bx1, bp0, bp1,
             inx0, inx1, inp0, inp1, out0, out1):
    srange = (s_hi - s_lo) // NW
    nch = srange // CHS
    wid = lax.axis_index("s") * NC + lax.axis_index("c")
    sbase = s_lo + wid * srange
    bxs = (bx0, bx1)
    bps = (bp0, bp1)
    inx = (inx0, inx1)
    inp = (inp0, inp1)
    outs = (out0, out1)

    def issue_loads(c, buf):
        soff = sbase + c * CHS
        lx = [pltpu.async_copy(x_hbm.at[b, pl.ds(soff, CHS)],
                               bxs[buf].at[b], inx[buf])
              for b in range(B)]
        lp = pltpu.async_copy(pos_hbm.at[pl.ds(soff, CHS)], bps[buf], inp[buf])
        return lx + [lp]

    def issue_store(c, buf):
        soff = sbase - s_lo + c * CHS
        return [pltpu.async_copy(bxs[buf].at[b],
                                 o_hbm.at[b, pl.ds(soff, CHS)], outs[buf])
                for b in range(B)]

    loads = [None] * nch
    stores = [None] * nch
    loads[0] = issue_loads(0, 0)
    for c in range(nch):
        cur = c % 2
        nxt = (c + 1) % 2
        if c + 1 < nch:
            if c - 1 >= 0:
                for d in stores[c - 1]:
                    d.wait()
            loads[c + 1] = issue_loads(c + 1, nxt)
        for d in loads[c]:
            d.wait()

        @pl.loop(0, CHS)
        def _(r):
            X = bxs[cur]
            P = bps[cur]
            for j in range(D // NLANES):
                sl = pl.ds(j * NLANES, NLANES)
                p = P.at[pl.ds(r, 1), sl][...]
                for b in range(B):
                    plsc.addupdate(X.at[b, pl.ds(r, 1), sl], p)

        stores[c] = issue_store(c, cur)
    for c in (nch - 2, nch - 1):
        if 0 <= c < nch:
            for d in stores[c]:
                d.wait()


@functools.lru_cache(maxsize=None)
def _make_sc_call(B, S, D, s_lo, s_hi):
    mesh = plsc.VectorSubcoreMesh(core_axis_name="c", subcore_axis_name="s")
    return pl.kernel(
        functools.partial(_sc_body, B, S, D, s_lo, s_hi),
        out_type=jax.ShapeDtypeStruct((B, s_hi - s_lo, D), jnp.float32),
        mesh=mesh,
        scratch_types=[
            pltpu.VMEM((B, CHS, D), jnp.float32),
            pltpu.VMEM((B, CHS, D), jnp.float32),
            pltpu.VMEM((CHS, D), jnp.float32),
            pltpu.VMEM((CHS, D), jnp.float32),
            pltpu.SemaphoreType.DMA,
            pltpu.SemaphoreType.DMA,
            pltpu.SemaphoreType.DMA,
            pltpu.SemaphoreType.DMA,
            pltpu.SemaphoreType.DMA,
            pltpu.SemaphoreType.DMA,
        ],
    )


def _tc_block_body(x_ref, pos_ref, o_ref):
    o_ref[...] = x_ref[...] + pos_ref[...][None, :, :]


@functools.lru_cache(maxsize=None)
def _make_tc_call(B, S, D, s_hi):
    return pl.pallas_call(
        _tc_block_body,
        grid=(s_hi // TC_BS,),
        in_specs=[
            pl.BlockSpec((B, TC_BS, D), lambda i: (0, i, 0)),
            pl.BlockSpec((TC_BS, D), lambda i: (i, 0)),
        ],
        out_specs=pl.BlockSpec((B, TC_BS, D), lambda i: (0, i, 0)),
        out_shape=jax.ShapeDtypeStruct((B, s_hi, D), jnp.float32),
    )


def kernel(x, pos_table):
    B, S, D = x.shape
    s_split = S // 2
    sc_out = _make_sc_call(B, S, D, s_split, S)(x, pos_table)
    tc_out = _make_tc_call(B, S, D, s_split)(x, pos_table)
    return jnp.concatenate([tc_out, sc_out], axis=1)


# speedup vs baseline: 2.4312x; 2.4312x over previous
import functools

import jax
import jax.numpy as jnp
from jax.experimental import pallas as pl


def _body(x_ref, pos_ref, o_ref):
    o_ref[...] = x_ref[...] + pos_ref[...][None, :, :]


def kernel(x, pos_table):
    B, S, D = x.shape
    BS = 512
    return pl.pallas_call(
        _body,
        grid=(S // BS, B),
        in_specs=[
            pl.BlockSpec((1, BS, D), lambda i, b: (b, i, 0)),
            pl.BlockSpec((BS, D), lambda i, b: (i, 0)),
        ],
        out_specs=pl.BlockSpec((1, BS, D), lambda i, b: (b, i, 0)),
        out_shape=jax.ShapeDtypeStruct((B, S, D), x.dtype),
    )(x, pos_table)


# speedup vs baseline: 2.4891x; 1.0238x over previous
import jax
import jax.numpy as jnp
from jax.experimental import pallas as pl


def _body(x_ref, pos_ref, o_ref):
    o_ref[...] = x_ref[...] + pos_ref[...][None, :, :]


def kernel(x, pos_table):
    B, S, D = x.shape
    BS = 128
    return pl.pallas_call(
        _body,
        grid=(S // BS,),
        in_specs=[
            pl.BlockSpec((B, BS, D), lambda i: (0, i, 0)),
            pl.BlockSpec((BS, D), lambda i: (i, 0)),
        ],
        out_specs=pl.BlockSpec((B, BS, D), lambda i: (0, i, 0)),
        out_shape=jax.ShapeDtypeStruct((B, S, D), x.dtype),
    )(x, pos_table)


# speedup vs baseline: 2.6983x; 1.0841x over previous
import jax
import jax.numpy as jnp
from jax.experimental import pallas as pl


def _body(x_ref, pos_ref, o_ref):
    o_ref[...] = x_ref[...] + pos_ref[...][None, :, :]


def kernel(x, pos_table):
    B, S, D = x.shape
    BS = 512
    return pl.pallas_call(
        _body,
        grid=(S // BS,),
        in_specs=[
            pl.BlockSpec((B, BS, D), lambda i: (0, i, 0)),
            pl.BlockSpec((BS, D), lambda i: (i, 0)),
        ],
        out_specs=pl.BlockSpec((B, BS, D), lambda i: (0, i, 0)),
        out_shape=jax.ShapeDtypeStruct((B, S, D), x.dtype),
    )(x, pos_table)
